# baseline (speedup 1.0000x reference)
import jax
import jax.numpy as jnp
from jax.experimental import pallas as pl
from jax.experimental.pallas import tpu as pltpu

N_E = 8192
E_DIM = 32
BETA = 0.25
TOKENS = 8192
BLK = 128
GRID = TOKENS // BLK
HALF = N_E // 2


def _bf16_rne(x):
    u = jax.lax.bitcast_convert_type(x, jnp.uint32)
    lower = u & jnp.uint32(0xFFFF)
    upper = u >> 16
    rb = ((lower > jnp.uint32(0x8000)) |
          ((lower == jnp.uint32(0x8000)) & ((upper & 1) == 1)))
    return jax.lax.bitcast_convert_type(
        (upper + rb.astype(jnp.uint32)) << 16, jnp.float32)


def _argmin_first(dd):
    m = jnp.min(dd, axis=1, keepdims=True)
    col = jax.lax.broadcasted_iota(jnp.int32, dd.shape, 1)
    i = jnp.min(jnp.where(dd == m, col, dd.shape[1]), axis=1)
    return m[:, 0], i


def _vq_body(z_ref, e_ref, oh_ref, idx_ref, zq_ref, loss_ref, perp_ref,
             cnt_ref, s_ref):
    i = pl.program_id(0)
    zb = z_ref[...]
    e = e_ref[...]

    sq = zb * zb
    a = sq[:, 0:1]
    for k in range(1, E_DIM):
        a = a + sq[:, k:k + 1]

    b = jnp.sum(e * e, axis=1)
    mm = jax.lax.dot_general(zb, e, (((1,), (1,)), ((), ())),
                             preferred_element_type=jnp.float32)
    d = a + b - 2.0 * mm

    m1, i1 = _argmin_first(d[:, :HALF])
    m2, i2 = _argmin_first(d[:, HALF:])
    take2 = m2 < _bf16_rne(m1)
    idx = jnp.where(take2, i2 + HALF, i1)
    idx_ref[...] = idx[:, None]

    col = jax.lax.broadcasted_iota(jnp.int32, (BLK, N_E), 1)
    oh = (col == idx[:, None]).astype(jnp.float32)
    oh_ref[...] = oh
    zq = jax.lax.dot_general(oh, e, (((1,), (0,)), ((), ())),
                             preferred_element_type=jnp.float32)
    t = zq - zb
    zq_ref[...] = zb + t
    ls = jnp.sum(t * t)
    cnt = jnp.sum(oh, axis=0)[None, :]

    @pl.when(i == 0)
    def _():
        s_ref[0, 0] = ls
        cnt_ref[...] = cnt

    @pl.when(i > 0)
    def _():
        s_ref[0, 0] = s_ref[0, 0] + ls
        cnt_ref[...] = cnt_ref[...] + cnt

    @pl.when(i == GRID - 1)
    def _():
        m = s_ref[0, 0] / (TOKENS * E_DIM)
        loss_ref[...] = jnp.reshape(m + BETA * m, (1, 1))
        em = cnt_ref[...] / TOKENS
        ent = jnp.sum(em * jnp.log(em + 1e-10))
        perp_ref[...] = jnp.reshape(jnp.exp(-ent), (1, 1))


def _vq_call(z_flat, embedding):
    return pl.pallas_call(
        _vq_body,
        grid=(GRID,),
        in_specs=[
            pl.BlockSpec((BLK, E_DIM), lambda i: (i, 0)),
            pl.BlockSpec((N_E, E_DIM), lambda i: (0, 0)),
        ],
        out_specs=(
            pl.BlockSpec((BLK, N_E), lambda i: (i, 0)),
            pl.BlockSpec((BLK, 1), lambda i: (i, 0)),
            pl.BlockSpec((BLK, E_DIM), lambda i: (i, 0)),
            pl.BlockSpec((1, 1), lambda i: (0, 0)),
            pl.BlockSpec((1, 1), lambda i: (0, 0)),
        ),
        out_shape=(
            jax.ShapeDtypeStruct((TOKENS, N_E), jnp.float32),
            jax.ShapeDtypeStruct((TOKENS, 1), jnp.int32),
            jax.ShapeDtypeStruct((TOKENS, E_DIM), jnp.float32),
            jax.ShapeDtypeStruct((1, 1), jnp.float32),
            jax.ShapeDtypeStruct((1, 1), jnp.float32),
        ),
        scratch_shapes=[
            pltpu.VMEM((1, N_E), jnp.float32),
            pltpu.SMEM((1, 1), jnp.float32),
        ],
        compiler_params=pltpu.CompilerParams(
            dimension_semantics=("arbitrary",)),
    )(z_flat, embedding)


def kernel(z, embedding):
    zp = jnp.transpose(z, (0, 2, 3, 1))
    z_flat = zp.reshape(-1, E_DIM)
    oh, idx, zq_st, loss, perp = _vq_call(z_flat, embedding)
    z_q_out = jnp.transpose(zq_st.reshape(zp.shape), (0, 3, 1, 2))
    return (loss[0, 0], z_q_out, perp[0, 0], oh, idx)


# speedup vs baseline: 1.0041x; 1.0041x over previous
import jax
import jax.numpy as jnp
from jax.experimental import pallas as pl
from jax.experimental.pallas import tpu as pltpu

N_E = 8192
E_DIM = 32
BETA = 0.25
TOKENS = 8192
BLK = 256
GRID = TOKENS // BLK
HALF = N_E // 2


def _bf16_rne(x):
    u = jax.lax.bitcast_convert_type(x, jnp.uint32)
    lower = u & jnp.uint32(0xFFFF)
    upper = u >> 16
    rb = ((lower > jnp.uint32(0x8000)) |
          ((lower == jnp.uint32(0x8000)) & ((upper & 1) == 1)))
    return jax.lax.bitcast_convert_type(
        (upper + rb.astype(jnp.uint32)) << 16, jnp.float32)


def _argmin_first(dd):
    m = jnp.min(dd, axis=1, keepdims=True)
    col = jax.lax.broadcasted_iota(jnp.int32, dd.shape, 1)
    i = jnp.min(jnp.where(dd == m, col, dd.shape[1]), axis=1)
    return m[:, 0], i


def _vq_body(z_ref, e_ref, oh_ref, idx_ref, zq_ref, loss_ref, perp_ref,
             cnt_ref, s_ref):
    i = pl.program_id(0)
    zb = z_ref[...]
    e = e_ref[...]

    sq = zb * zb
    a = sq[:, 0:1]
    for k in range(1, E_DIM):
        a = a + sq[:, k:k + 1]

    b = jnp.sum(e * e, axis=1)
    mm = jax.lax.dot_general(zb, e, (((1,), (1,)), ((), ())),
                             preferred_element_type=jnp.float32)
    d = a + b - 2.0 * mm

    m1, i1 = _argmin_first(d[:, :HALF])
    m2, i2 = _argmin_first(d[:, HALF:])
    take2 = m2 < _bf16_rne(m1)
    idx = jnp.where(take2, i2 + HALF, i1)
    idx_ref[...] = idx[:, None]

    col = jax.lax.broadcasted_iota(jnp.int32, (BLK, N_E), 1)
    oh = (col == idx[:, None]).astype(jnp.float32)
    oh_ref[...] = oh
    zq = jax.lax.dot_general(oh, e, (((1,), (0,)), ((), ())),
                             preferred_element_type=jnp.float32)
    t = zq - zb
    zq_ref[...] = zb + t
    ls = jnp.sum(t * t)
    cnt = jnp.sum(oh, axis=0)[None, :]

    @pl.when(i == 0)
    def _():
        s_ref[0, 0] = ls
        cnt_ref[...] = cnt

    @pl.when(i > 0)
    def _():
        s_ref[0, 0] = s_ref[0, 0] + ls
        cnt_ref[...] = cnt_ref[...] + cnt

    @pl.when(i == GRID - 1)
    def _():
        m = s_ref[0, 0] / (TOKENS * E_DIM)
        loss_ref[...] = jnp.reshape(m + BETA * m, (1, 1))
        em = cnt_ref[...] / TOKENS
        ent = jnp.sum(em * jnp.log(em + 1e-10))
        perp_ref[...] = jnp.reshape(jnp.exp(-ent), (1, 1))


def _vq_call(z_flat, embedding):
    return pl.pallas_call(
        _vq_body,
        grid=(GRID,),
        in_specs=[
            pl.BlockSpec((BLK, E_DIM), lambda i: (i, 0)),
            pl.BlockSpec((N_E, E_DIM), lambda i: (0, 0)),
        ],
        out_specs=(
            pl.BlockSpec((BLK, N_E), lambda i: (i, 0)),
            pl.BlockSpec((BLK, 1), lambda i: (i, 0)),
            pl.BlockSpec((BLK, E_DIM), lambda i: (i, 0)),
            pl.BlockSpec((1, 1), lambda i: (0, 0)),
            pl.BlockSpec((1, 1), lambda i: (0, 0)),
        ),
        out_shape=(
            jax.ShapeDtypeStruct((TOKENS, N_E), jnp.float32),
            jax.ShapeDtypeStruct((TOKENS, 1), jnp.int32),
            jax.ShapeDtypeStruct((TOKENS, E_DIM), jnp.float32),
            jax.ShapeDtypeStruct((1, 1), jnp.float32),
            jax.ShapeDtypeStruct((1, 1), jnp.float32),
        ),
        scratch_shapes=[
            pltpu.VMEM((1, N_E), jnp.float32),
            pltpu.SMEM((1, 1), jnp.float32),
        ],
        compiler_params=pltpu.CompilerParams(
            dimension_semantics=("arbitrary",)),
    )(z_flat, embedding)


def kernel(z, embedding):
    zp = jnp.transpose(z, (0, 2, 3, 1))
    z_flat = zp.reshape(-1, E_DIM)
    oh, idx, zq_st, loss, perp = _vq_call(z_flat, embedding)
    z_q_out = jnp.transpose(zq_st.reshape(zp.shape), (0, 3, 1, 2))
    return (loss[0, 0], z_q_out, perp[0, 0], oh, idx)
